# Initial kernel scaffold; baseline (speedup 1.0000x reference)
import jax
import jax.numpy as jnp
from jax.experimental import pallas as pl


def kernel(val, global_token, pos_emb, remain_idx):
    raise NotImplementedError("write your pallas kernel here")


# speedup vs baseline: 4.3247x; 4.3247x over previous
import functools

import jax
import jax.numpy as jnp
from jax import lax
from jax.experimental import pallas as pl
from jax.experimental.pallas import tpu as pltpu
from jax.experimental.pallas import tpu_sc as plsc

B, S, D = 16, 2048, 1024
REMAIN = 1024
NC, NS, L = 2, 16, 16
NW = NC * NS
RPW = (B * REMAIN) // NW
K = 32
NCHUNK = RPW // K
OUT_ROWS = B * (REMAIN + 1)


def _sc_body(val_hbm, gt_hbm, pos_hbm, idx_hbm, out_hbm,
             idx_b, iv_b, ip_b, vbuf, pbuf, sem_v, sem_p):
    cid = lax.axis_index("c")
    sid = lax.axis_index("s")
    w = sid * NC + cid
    b = w // 2
    h = w % 2
    base = w * RPW
    out_base = b * (REMAIN + 1) + 1 + h * RPW

    pltpu.sync_copy(idx_hbm.at[pl.ds(base, RPW)], idx_b)

    def ixbody(i, _):
        v = idx_b[pl.ds(i * L, L)]
        iv_b[pl.ds(i * L, L)] = v + b * S
        ip_b[pl.ds(i * L, L)] = v + 1
        return 0

    lax.fori_loop(0, RPW // L, ixbody, 0)

    @pl.when(h == 0)
    def _():
        pltpu.sync_copy(gt_hbm, vbuf.at[pl.ds(0, 1)])
        pltpu.sync_copy(pos_hbm.at[pl.ds(0, 1)], pbuf.at[pl.ds(0, 1)])

        def gtbody(i, _):
            vbuf[0, pl.ds(i * L, L)] = (vbuf[0, pl.ds(i * L, L)]
                                        + pbuf[0, pl.ds(i * L, L)])
            return 0

        lax.fori_loop(0, D // L, gtbody, 0)
        pltpu.sync_copy(vbuf.at[pl.ds(0, 1)],
                        out_hbm.at[pl.ds(b * (REMAIN + 1), 1)])

    def chunk(cix, _):
        iv = iv_b.at[pl.ds(cix * K, K)]
        ip = ip_b.at[pl.ds(cix * K, K)]
        cp_v = pltpu.async_copy(val_hbm.at[iv], vbuf, sem_v)
        cp_p = pltpu.async_copy(pos_hbm.at[ip], pbuf, sem_p)
        cp_v.wait()
        cp_p.wait()

        def addrow(r, _):
            for cc in range(D // L):
                vbuf[r, pl.ds(cc * L, L)] = (vbuf[r, pl.ds(cc * L, L)]
                                             + pbuf[r, pl.ds(cc * L, L)])
            return 0

        lax.fori_loop(0, K, addrow, 0)
        pltpu.sync_copy(vbuf, out_hbm.at[pl.ds(out_base + cix * K, K)])
        return 0

    lax.fori_loop(0, NCHUNK, chunk, 0)


@jax.jit
def _sc_gather(valf, gt, pos, idx):
    mesh = plsc.VectorSubcoreMesh(core_axis_name="c", subcore_axis_name="s",
                                  num_cores=NC, num_subcores=NS)
    fn = pl.kernel(
        _sc_body,
        out_type=jax.ShapeDtypeStruct((OUT_ROWS, D), jnp.float32),
        mesh=mesh,
        scratch_types=[
            pltpu.VMEM((RPW,), jnp.int32),
            pltpu.VMEM((RPW,), jnp.int32),
            pltpu.VMEM((RPW,), jnp.int32),
            pltpu.VMEM((K, D), jnp.float32),
            pltpu.VMEM((K, D), jnp.float32),
            pltpu.SemaphoreType.DMA,
            pltpu.SemaphoreType.DMA,
        ],
        compiler_params=pltpu.CompilerParams(use_tc_tiling_on_sc=False),
    )
    return fn(valf, gt, pos, idx)


def kernel(val, global_token, pos_emb, remain_idx):
    idx = remain_idx.astype(jnp.int32).reshape(B * REMAIN)
    valf = val.reshape(B * S, D)
    out = _sc_gather(valf, global_token, pos_emb, idx)
    return out.reshape(B, REMAIN + 1, D)


# speedup vs baseline: 4.9059x; 1.1344x over previous
import functools

import jax
import jax.numpy as jnp
from jax import lax
from jax.experimental import pallas as pl
from jax.experimental.pallas import tpu as pltpu
from jax.experimental.pallas import tpu_sc as plsc

B, S, D = 16, 2048, 1024
REMAIN = 1024
NC, NS, L = 2, 16, 16
NW = NC * NS
RPW = (B * REMAIN) // NW
K = 16
NCHUNK = RPW // K
OUT_ROWS = B * (REMAIN + 1)


def _sc_body(val_hbm, gt_hbm, pos_hbm, idx_hbm, out_hbm,
             idx_b, iv_b, ip_b, vbuf0, pbuf0, vbuf1, pbuf1,
             sem_v0, sem_p0, sem_v1, sem_p1):
    cid = lax.axis_index("c")
    sid = lax.axis_index("s")
    w = sid * NC + cid
    b = w // 2
    h = w % 2
    base = w * RPW
    out_base = b * (REMAIN + 1) + 1 + h * RPW

    pltpu.sync_copy(idx_hbm.at[pl.ds(base, RPW)], idx_b)

    def ixbody(i, _):
        v = idx_b[pl.ds(i * L, L)]
        iv_b[pl.ds(i * L, L)] = v + b * S
        ip_b[pl.ds(i * L, L)] = v + 1
        return 0

    lax.fori_loop(0, RPW // L, ixbody, 0)

    def fetch(cix, vbuf, pbuf, sem_v, sem_p):
        iv = iv_b.at[pl.ds(cix * K, K)]
        ip = ip_b.at[pl.ds(cix * K, K)]
        pltpu.async_copy(val_hbm.at[iv], vbuf, sem_v)
        pltpu.async_copy(pos_hbm.at[ip], pbuf, sem_p)

    def drain(cix, vbuf, pbuf, sem_v, sem_p):
        iv = iv_b.at[pl.ds(cix * K, K)]
        ip = ip_b.at[pl.ds(cix * K, K)]
        pltpu.make_async_copy(val_hbm.at[iv], vbuf, sem_v).wait()
        pltpu.make_async_copy(pos_hbm.at[ip], pbuf, sem_p).wait()

    def process(cix, vbuf, pbuf):
        def addrow(r, _):
            for cc in range(D // L):
                plsc.addupdate(vbuf.at[r, pl.ds(cc * L, L)],
                               pbuf[r, pl.ds(cc * L, L)])
            return 0

        lax.fori_loop(0, K, addrow, 0)
        pltpu.sync_copy(vbuf, out_hbm.at[pl.ds(out_base + cix * K, K)])

    fetch(0, vbuf0, pbuf0, sem_v0, sem_p0)

    slots = ((vbuf0, pbuf0, sem_v0, sem_p0), (vbuf1, pbuf1, sem_v1, sem_p1))

    def pair(i, _):
        for sub in range(2):
            cix = i * 2 + sub
            vb, pb, sv, sp = slots[sub]
            nvb, npb, nsv, nsp = slots[1 - sub]
            drain(cix, vb, pb, sv, sp)

            @pl.when(cix + 1 < NCHUNK)
            def _():
                fetch(cix + 1, nvb, npb, nsv, nsp)

            process(cix, vb, pb)
        return 0

    lax.fori_loop(0, NCHUNK // 2, pair, 0)

    @pl.when(h == 0)
    def _():
        pltpu.sync_copy(gt_hbm, vbuf0.at[pl.ds(0, 1)])
        pltpu.sync_copy(pos_hbm.at[pl.ds(0, 1)], pbuf0.at[pl.ds(0, 1)])

        def gtbody(i, _):
            plsc.addupdate(vbuf0.at[0, pl.ds(i * L, L)],
                           pbuf0[0, pl.ds(i * L, L)])
            return 0

        lax.fori_loop(0, D // L, gtbody, 0)
        pltpu.sync_copy(vbuf0.at[pl.ds(0, 1)],
                        out_hbm.at[pl.ds(b * (REMAIN + 1), 1)])


@jax.jit
def _sc_gather(valf, gt, pos, idx):
    mesh = plsc.VectorSubcoreMesh(core_axis_name="c", subcore_axis_name="s",
                                  num_cores=NC, num_subcores=NS)
    fn = pl.kernel(
        _sc_body,
        out_type=jax.ShapeDtypeStruct((OUT_ROWS, D), jnp.float32),
        mesh=mesh,
        scratch_types=[
            pltpu.VMEM((RPW,), jnp.int32),
            pltpu.VMEM((RPW,), jnp.int32),
            pltpu.VMEM((RPW,), jnp.int32),
            pltpu.VMEM((K, D), jnp.float32),
            pltpu.VMEM((K, D), jnp.float32),
            pltpu.VMEM((K, D), jnp.float32),
            pltpu.VMEM((K, D), jnp.float32),
            pltpu.SemaphoreType.DMA,
            pltpu.SemaphoreType.DMA,
            pltpu.SemaphoreType.DMA,
            pltpu.SemaphoreType.DMA,
        ],
        compiler_params=pltpu.CompilerParams(use_tc_tiling_on_sc=False),
    )
    return fn(valf, gt, pos, idx)


def kernel(val, global_token, pos_emb, remain_idx):
    idx = remain_idx.astype(jnp.int32).reshape(B * REMAIN)
    valf = val.reshape(B * S, D)
    out = _sc_gather(valf, global_token, pos_emb, idx)
    return out.reshape(B, REMAIN + 1, D)


# speedup vs baseline: 9.7958x; 1.9967x over previous
import functools

import jax
import jax.numpy as jnp
from jax import lax
from jax.experimental import pallas as pl
from jax.experimental.pallas import tpu as pltpu
from jax.experimental.pallas import tpu_sc as plsc

B, S, D = 16, 2048, 1024
REMAIN = 1024
NC, NS, L = 2, 16, 16
NW = NC * NS
RPW = (B * REMAIN) // NW
K = 16
NCHUNK = RPW // K


def _sc_body(val_hbm, gt_hbm, pos_hbm, idx_hbm, out_hbm,
             idx_b, iv_b, ip_b, vbuf0, pbuf0, vbuf1, pbuf1,
             sem_v0, sem_p0, sem_v1, sem_p1, sem_o):
    cid = lax.axis_index("c")
    sid = lax.axis_index("s")
    w = sid * NC + cid
    b = w // 2
    h = w % 2
    base = w * RPW
    out_b = out_hbm.at[b]
    out_base = 1 + h * RPW
    lane = lax.broadcasted_iota(jnp.int32, (L,), 0)

    pltpu.sync_copy(idx_hbm.at[pl.ds(base, RPW)], idx_b)

    def ixbody(i, _):
        v = idx_b[pl.ds(i * L, L)]
        iv_b[pl.ds(i * L, L)] = v + b * S
        ip_b[pl.ds(i * L, L)] = v + 1
        return 0

    lax.fori_loop(0, RPW // L, ixbody, 0)

    def fetch(cix, vbuf, pbuf, sem_v, sem_p):
        iv = iv_b.at[pl.ds(cix * K, K)]
        ip = ip_b.at[pl.ds(cix * K, K)]
        pltpu.async_copy(val_hbm.at[iv], vbuf, sem_v)
        pltpu.async_copy(pos_hbm.at[ip], pbuf, sem_p)

    def drain(cix, vbuf, pbuf, sem_v, sem_p):
        iv = iv_b.at[pl.ds(cix * K, K)]
        ip = ip_b.at[pl.ds(cix * K, K)]
        pltpu.make_async_copy(val_hbm.at[iv], vbuf, sem_v).wait()
        pltpu.make_async_copy(pos_hbm.at[ip], pbuf, sem_p).wait()

    def process(cix, vbuf, pbuf):
        def addrow(r, _):
            for cc in range(D // L):
                plsc.addupdate(vbuf.at[r, pl.ds(cc * L, L)],
                               pbuf[r, pl.ds(cc * L, L)])
            return 0

        lax.fori_loop(0, K, addrow, 0)
        oi = out_base + cix * K + lane
        pltpu.async_copy(vbuf, out_b.at[oi], sem_o).wait()

    fetch(0, vbuf0, pbuf0, sem_v0, sem_p0)

    slots = ((vbuf0, pbuf0, sem_v0, sem_p0), (vbuf1, pbuf1, sem_v1, sem_p1))

    def pair(i, _):
        for sub in range(2):
            cix = i * 2 + sub
            vb, pb, sv, sp = slots[sub]
            nvb, npb, nsv, nsp = slots[1 - sub]
            drain(cix, vb, pb, sv, sp)

            @pl.when(cix + 1 < NCHUNK)
            def _():
                fetch(cix + 1, nvb, npb, nsv, nsp)

            process(cix, vb, pb)
        return 0

    lax.fori_loop(0, NCHUNK // 2, pair, 0)

    @pl.when(h == 0)
    def _():
        pltpu.sync_copy(gt_hbm, vbuf0.at[pl.ds(0, 1)])
        pltpu.sync_copy(pos_hbm.at[pl.ds(0, 1)], pbuf0.at[pl.ds(0, 1)])

        def gtadd(i, _):
            plsc.addupdate(vbuf0.at[0, pl.ds(i * L, L)],
                           pbuf0[0, pl.ds(i * L, L)])
            return 0

        lax.fori_loop(0, D // L, gtadd, 0)

        def gtdup(r, _):
            for cc in range(D // L):
                vbuf0[r, pl.ds(cc * L, L)] = vbuf0[0, pl.ds(cc * L, L)]
            return 0

        lax.fori_loop(1, L, gtdup, 0)
        pltpu.async_copy(vbuf0, out_b.at[lane * 0], sem_o).wait()


@jax.jit
def _sc_gather(valf, gt, pos, idx):
    mesh = plsc.VectorSubcoreMesh(core_axis_name="c", subcore_axis_name="s",
                                  num_cores=NC, num_subcores=NS)
    fn = pl.kernel(
        _sc_body,
        out_type=jax.ShapeDtypeStruct((B, REMAIN + 1, D), jnp.float32),
        mesh=mesh,
        scratch_types=[
            pltpu.VMEM((RPW,), jnp.int32),
            pltpu.VMEM((RPW,), jnp.int32),
            pltpu.VMEM((RPW,), jnp.int32),
            pltpu.VMEM((K, D), jnp.float32),
            pltpu.VMEM((K, D), jnp.float32),
            pltpu.VMEM((K, D), jnp.float32),
            pltpu.VMEM((K, D), jnp.float32),
            pltpu.SemaphoreType.DMA,
            pltpu.SemaphoreType.DMA,
            pltpu.SemaphoreType.DMA,
            pltpu.SemaphoreType.DMA,
            pltpu.SemaphoreType.DMA,
        ],
    )
    return fn(valf, gt, pos, idx)


def kernel(val, global_token, pos_emb, remain_idx):
    idx = remain_idx.astype(jnp.int32).reshape(B * REMAIN)
    valf = val.reshape(B * S, D)
    return _sc_gather(valf, global_token, pos_emb, idx)


# speedup vs baseline: 15.5634x; 1.5888x over previous
import jax
import jax.numpy as jnp
from jax import lax
from jax.experimental import pallas as pl
from jax.experimental.pallas import tpu as pltpu
from jax.experimental.pallas import tpu_sc as plsc

B, S, D = 16, 2048, 1024
REMAIN = 1024
NC, NS, L = 2, 16, 16
NW = NC * NS
RPW = (B * REMAIN) // NW
K = 16
NCHUNK = RPW // K


def _sc_body(val_hbm, gt_hbm, pos_hbm, idx_hbm, out_hbm,
             idx_b, iv_b, ip_b, vbuf0, pbuf0, vbuf1, pbuf1, vbuf2, pbuf2,
             gbuf, sem_v0, sem_p0, sem_v1, sem_p1, sem_v2, sem_p2,
             sem_o, sem_g):
    cid = lax.axis_index("c")
    sid = lax.axis_index("s")
    w = sid * NC + cid
    b = w // 2
    h = w % 2
    out_base = (1 + h * RPW) * B + b
    lane = lax.broadcasted_iota(jnp.int32, (L,), 0)

    pltpu.sync_copy(idx_hbm.at[b // 8, pl.ds(h * 4, 4), b % 8], idx_b)

    is_gt_worker = h == b % 2

    @pl.when(is_gt_worker)
    def _():
        pltpu.async_copy(gt_hbm, gbuf.at[pl.ds(0, 1)], sem_g)
        pltpu.async_copy(pos_hbm.at[pl.ds(0, 1)], gbuf.at[pl.ds(1, 1)], sem_g)

    def ixbody(i, _):
        v = idx_b[i // 8, pl.ds((i % 8) * L, L)]
        iv_b[pl.ds(i * L, L)] = v + b * S
        ip_b[pl.ds(i * L, L)] = v + 1
        return 0

    lax.fori_loop(0, RPW // L, ixbody, 0)

    def fetch(cix, vbuf, pbuf, sem_v, sem_p):
        iv = iv_b.at[pl.ds(cix * K, K)]
        ip = ip_b.at[pl.ds(cix * K, K)]
        pltpu.async_copy(val_hbm.at[iv], vbuf, sem_v)
        pltpu.async_copy(pos_hbm.at[ip], pbuf, sem_p)

    def drain(cix, vbuf, pbuf, sem_v, sem_p):
        iv = iv_b.at[pl.ds(cix * K, K)]
        ip = ip_b.at[pl.ds(cix * K, K)]
        pltpu.make_async_copy(val_hbm.at[iv], vbuf, sem_v).wait()
        pltpu.make_async_copy(pos_hbm.at[ip], pbuf, sem_p).wait()

    def process(cix, vbuf, pbuf):
        def addrow(r, _):
            for cc in range(D // L):
                plsc.addupdate(vbuf.at[r, pl.ds(cc * L, L)],
                               pbuf[r, pl.ds(cc * L, L)])
            return 0

        lax.fori_loop(0, K, addrow, 0)
        oi = out_base + (cix * K + lane) * B
        pltpu.async_copy(vbuf, out_hbm.at[oi], sem_o).wait()

    slots = ((vbuf0, pbuf0, sem_v0, sem_p0),
             (vbuf1, pbuf1, sem_v1, sem_p1),
             (vbuf2, pbuf2, sem_v2, sem_p2))
    RING = 3
    fetch(0, *slots[0])
    fetch(1, *slots[1])

    def trio(i, _):
        for sub in range(RING):
            cix = i * RING + sub
            drain(cix, *slots[sub])

            @pl.when(cix + 2 < NCHUNK)
            def _():
                fetch(cix + 2, *slots[(sub + 2) % RING])

            process(cix, *slots[sub][:2])
        return 0

    lax.fori_loop(0, NCHUNK // RING, trio, 0)
    for cix in range(NCHUNK - NCHUNK % RING, NCHUNK):
        drain(cix, *slots[cix % RING])
        process(cix, *slots[cix % RING][:2])

    @pl.when(is_gt_worker)
    def _():
        pltpu.make_async_copy(gt_hbm, gbuf.at[pl.ds(0, 1)], sem_g).wait()
        pltpu.make_async_copy(pos_hbm.at[pl.ds(0, 1)],
                              gbuf.at[pl.ds(1, 1)], sem_g).wait()

        def gtadd(i, _):
            plsc.addupdate(gbuf.at[0, pl.ds(i * L, L)],
                           gbuf[1, pl.ds(i * L, L)])
            return 0

        lax.fori_loop(0, D // L, gtadd, 0)

        def gtdup(r, _):
            for cc in range(D // L):
                vbuf0[r, pl.ds(cc * L, L)] = gbuf[0, pl.ds(cc * L, L)]
            return 0

        lax.fori_loop(0, L, gtdup, 0)
        pltpu.async_copy(vbuf0, out_hbm.at[lane * 0 + b], sem_o).wait()


@jax.jit
def _sc_gather(valf, gt, pos, idx):
    mesh = plsc.VectorSubcoreMesh(core_axis_name="c", subcore_axis_name="s",
                                  num_cores=NC, num_subcores=NS)
    fn = pl.kernel(
        _sc_body,
        out_type=jax.ShapeDtypeStruct(((REMAIN + 1) * B, D), jnp.float32),
        mesh=mesh,
        scratch_types=[
            pltpu.VMEM((RPW // 128, 128), jnp.int32),
            pltpu.VMEM((RPW,), jnp.int32),
            pltpu.VMEM((RPW,), jnp.int32),
            pltpu.VMEM((K, D), jnp.float32),
            pltpu.VMEM((K, D), jnp.float32),
            pltpu.VMEM((K, D), jnp.float32),
            pltpu.VMEM((K, D), jnp.float32),
            pltpu.VMEM((K, D), jnp.float32),
            pltpu.VMEM((K, D), jnp.float32),
            pltpu.VMEM((2, D), jnp.float32),
            pltpu.SemaphoreType.DMA,
            pltpu.SemaphoreType.DMA,
            pltpu.SemaphoreType.DMA,
            pltpu.SemaphoreType.DMA,
            pltpu.SemaphoreType.DMA,
            pltpu.SemaphoreType.DMA,
            pltpu.SemaphoreType.DMA,
            pltpu.SemaphoreType.DMA,
        ],
    )
    return fn(valf, gt, pos, idx)


def kernel(val, global_token, pos_emb, remain_idx):
    idx4 = (remain_idx.astype(jnp.int32)
            .reshape(B // 8, 8, REMAIN // 128, 128).transpose(0, 2, 1, 3))
    valf = val.reshape(B * S, D)
    out = _sc_gather(valf, global_token, pos_emb, idx4)
    return out.reshape(REMAIN + 1, B, D).transpose(1, 0, 2)
